# Initial kernel scaffold; baseline (speedup 1.0000x reference)
import jax
import jax.numpy as jnp
from jax.experimental import pallas as pl


def kernel(x, W_gate, Wg, Wu, Wd):
    raise NotImplementedError("write your pallas kernel here")


# speedup vs baseline: 1.5505x; 1.5505x over previous
import functools

import jax
import jax.numpy as jnp
from jax.experimental import pallas as pl
from jax.experimental.pallas import tpu as pltpu


def _moe_dense_kernel(x_ref, wgate_ref, wg_ref, wu_ref, wd_ref, o_ref):
    e = pl.program_id(1)
    f = pl.program_id(2)

    x = x_ref[...]

    logits = jax.lax.dot_general(
        x, wgate_ref[...], (((1,), (1,)), ((), ())),
        preferred_element_type=jnp.float32)
    eidx = jax.lax.broadcasted_iota(jnp.int32, logits.shape, 1)
    n_e = logits.shape[-1]
    m1 = jnp.max(logits, axis=-1, keepdims=True)
    i1 = jnp.min(jnp.where(logits == m1, eidx, n_e), axis=-1, keepdims=True)
    oh1 = (eidx == i1)
    masked = jnp.where(oh1, -jnp.inf, logits)
    m2 = jnp.max(masked, axis=-1, keepdims=True)
    i2 = jnp.min(jnp.where(masked == m2, eidx, n_e), axis=-1, keepdims=True)
    oh2 = (eidx == i2)
    z = jnp.exp(m2 - m1)
    p1 = 1.0 / (1.0 + z)
    p2 = z * p1
    sel1 = jnp.sum(jnp.where(jnp.logical_and(oh1, eidx == e), 1.0, 0.0),
                   axis=-1, keepdims=True)
    sel2 = jnp.sum(jnp.where(jnp.logical_and(oh2, eidx == e), 1.0, 0.0),
                   axis=-1, keepdims=True)
    w = p1 * sel1 + p2 * sel2

    g = jax.lax.dot_general(
        x, wg_ref[0], (((1,), (1,)), ((), ())),
        preferred_element_type=jnp.float32)
    u = jax.lax.dot_general(
        x, wu_ref[0], (((1,), (1,)), ((), ())),
        preferred_element_type=jnp.float32)
    h = g * jax.nn.sigmoid(g) * u
    out_p = jax.lax.dot_general(
        h, wd_ref[0], (((1,), (1,)), ((), ())),
        preferred_element_type=jnp.float32)
    acc = w * out_p

    @pl.when(jnp.logical_and(e == 0, f == 0))
    def _init():
        o_ref[...] = acc

    @pl.when(jnp.logical_not(jnp.logical_and(e == 0, f == 0)))
    def _acc():
        o_ref[...] = o_ref[...] + acc


def _moe_dense(x_flat, W_gate, Wg, Wu, Wd, *, bm, dff):
    T, H = x_flat.shape
    E, D_FF, _ = Wg.shape
    grid = (T // bm, E, D_FF // dff)
    return pl.pallas_call(
        _moe_dense_kernel,
        grid=grid,
        in_specs=[
            pl.BlockSpec((bm, H), lambda i, e, f: (i, 0)),
            pl.BlockSpec((E, H), lambda i, e, f: (0, 0)),
            pl.BlockSpec((1, dff, H), lambda i, e, f: (e, f, 0)),
            pl.BlockSpec((1, dff, H), lambda i, e, f: (e, f, 0)),
            pl.BlockSpec((1, H, dff), lambda i, e, f: (e, 0, f)),
        ],
        out_specs=pl.BlockSpec((bm, H), lambda i, e, f: (i, 0)),
        out_shape=jax.ShapeDtypeStruct((T, H), x_flat.dtype),
        compiler_params=pltpu.CompilerParams(
            dimension_semantics=("arbitrary", "arbitrary", "arbitrary"),
        ),
    )(x_flat, W_gate, Wg, Wu, Wd)


def kernel(x, W_gate, Wg, Wu, Wd):
    batch, seq, hidden = x.shape
    x_flat = x.reshape(-1, hidden)
    out = _moe_dense(x_flat, W_gate, Wg, Wu, Wd, bm=512, dff=1408)
    return out.reshape(batch, seq, hidden)
